# baseline (speedup 1.0000x reference)
import functools

import jax
import jax.numpy as jnp
from jax import lax
from jax.experimental import pallas as pl
from jax.experimental.pallas import tpu as pltpu
from jax.experimental.pallas import tpu_sc as plsc

T = 2048
D = 768
H = 4 * D
E = 8
K = 2
A = T * K

TILE_G = 512
NT = -(-A // TILE_G) + E
NP = NT * TILE_G
TILE_T = 256

NC, NS = 2, 16
NW = NC * NS
CH = T // NW

_sc_mesh = plsc.VectorSubcoreMesh(core_axis_name="c", subcore_axis_name="s")


def _router_body(x_ref, wr_ref, w_ref, idx_ref):
    logits = jnp.dot(x_ref[...], wr_ref[...], preferred_element_type=jnp.float32)
    m = jnp.max(logits, axis=-1, keepdims=True)
    el = jnp.exp(logits - m)
    probs = el / jnp.sum(el, axis=-1, keepdims=True)
    lane = jax.lax.broadcasted_iota(jnp.int32, probs.shape, 1)
    p1 = jnp.max(probs, axis=-1, keepdims=True)
    i1 = jnp.min(jnp.where(probs == p1, lane, E), axis=-1, keepdims=True)
    pm = jnp.where(lane == i1, -1.0, probs)
    p2 = jnp.max(pm, axis=-1, keepdims=True)
    i2 = jnp.min(jnp.where(pm == p2, lane, E), axis=-1, keepdims=True)
    w_ref[...] = jnp.concatenate([p1, p2], axis=-1)
    idx_ref[...] = jnp.concatenate([i1, i2], axis=-1)


@functools.partial(
    pl.kernel, mesh=_sc_mesh,
    out_type=jax.ShapeDtypeStruct((NP, D), jnp.float32),
    scratch_types=[
        pltpu.VMEM((CH,), jnp.int32),
        pltpu.VMEM((CH,), jnp.int32),
        pltpu.VMEM((CH, D), jnp.float32),
        pltpu.SemaphoreType.DMA,
    ],
)
def _sc_scatter_x(x_hbm, posk_hbm, xs_hbm, idx0_v, idx1_v, rows_v, sem):
    wid = lax.axis_index("s") * NC + lax.axis_index("c")
    base = wid * CH
    pltpu.sync_copy(posk_hbm.at[0, pl.ds(base, CH)], idx0_v)
    pltpu.sync_copy(posk_hbm.at[1, pl.ds(base, CH)], idx1_v)
    pltpu.sync_copy(x_hbm.at[pl.ds(base, CH)], rows_v)
    pltpu.async_copy(rows_v, xs_hbm.at[idx0_v], sem).wait()
    pltpu.async_copy(rows_v, xs_hbm.at[idx1_v], sem).wait()


@functools.partial(
    pl.kernel, mesh=_sc_mesh,
    out_type=jax.ShapeDtypeStruct((2 * T, D), jnp.float32),
    scratch_types=[
        pltpu.VMEM((CH,), jnp.int32),
        pltpu.VMEM((CH, D), jnp.float32),
        pltpu.SemaphoreType.DMA,
    ],
)
def _sc_gather_back(eo_hbm, posk_hbm, g_hbm, idx_v, rows_v, sem):
    wid = lax.axis_index("s") * NC + lax.axis_index("c")
    base = wid * CH
    pltpu.sync_copy(posk_hbm.at[0, pl.ds(base, CH)], idx_v)
    pltpu.async_copy(eo_hbm.at[idx_v], rows_v, sem).wait()
    pltpu.sync_copy(rows_v, g_hbm.at[pl.ds(base, CH)])
    pltpu.sync_copy(posk_hbm.at[1, pl.ds(base, CH)], idx_v)
    pltpu.async_copy(eo_hbm.at[idx_v], rows_v, sem).wait()
    pltpu.sync_copy(rows_v, g_hbm.at[pl.ds(T + base, CH)])


def _ffn_body(te_ref, nact_ref, x_ref, w1_ref, b1_ref, w2_ref, b2_ref, out_ref,
              w1b_ref, w2b_ref):
    g = pl.program_id(0)
    prev_e = jnp.where(g == 0, -1, te_ref[jnp.maximum(g - 1, 0)])

    @pl.when((g < nact_ref[0]) & (te_ref[g] != prev_e))
    def _cast_weights():
        w1b_ref[...] = w1_ref[0].astype(jnp.bfloat16)
        w2b_ref[...] = w2_ref[0].astype(jnp.bfloat16)

    @pl.when(g < nact_ref[0])
    def _compute():
        xb = x_ref[...].astype(jnp.bfloat16)
        h = jnp.dot(xb, w1b_ref[...], preferred_element_type=jnp.float32)
        h = h + b1_ref[0]
        h = h * (1.0 / (1.0 + jnp.exp(-h)))
        out_ref[...] = jnp.dot(h.astype(jnp.bfloat16), w2b_ref[...],
                               preferred_element_type=jnp.float32) + b2_ref[0]


def _combine_body(x_ref, g1_ref, g2_ref, w_ref, gamma_ref, beta_ref, gs_ref,
                  out_ref):
    w = w_ref[...]
    out = w[:, 0:1] * g1_ref[...] + w[:, 1:2] * g2_ref[...]
    y = x_ref[...] + out * gs_ref[0]
    mu = jnp.mean(y, axis=-1, keepdims=True)
    yc = y - mu
    var = jnp.mean(yc * yc, axis=-1, keepdims=True)
    out_ref[...] = yc * jax.lax.rsqrt(var + 1e-5) * gamma_ref[...] + beta_ref[...]


@jax.jit
def kernel(x, W_r, W1, b1, W2, b2, gamma, beta, gate_scale):
    w_top, idx_top = pl.pallas_call(
        _router_body,
        grid=(T // TILE_T,),
        in_specs=[
            pl.BlockSpec((TILE_T, D), lambda t: (t, 0)),
            pl.BlockSpec((D, E), lambda t: (0, 0)),
        ],
        out_specs=[
            pl.BlockSpec((TILE_T, K), lambda t: (t, 0)),
            pl.BlockSpec((TILE_T, K), lambda t: (t, 0)),
        ],
        out_shape=[
            jax.ShapeDtypeStruct((T, K), jnp.float32),
            jax.ShapeDtypeStruct((T, K), jnp.int32),
        ],
    )(x, W_r)

    ef = idx_top.reshape(-1)
    oh = (ef[:, None] == jnp.arange(E, dtype=jnp.int32)[None, :]).astype(jnp.int32)
    csum = jnp.cumsum(oh, axis=0)
    counts = csum[-1]
    rank = jnp.sum(csum * oh, axis=1) - 1
    padded = ((counts + TILE_G - 1) // TILE_G) * TILE_G
    base = jnp.concatenate([jnp.zeros((1,), jnp.int32),
                            jnp.cumsum(padded)[:-1].astype(jnp.int32)])
    pos = base[ef] + rank
    posk = pos.reshape(T, K).T
    ends = base + padded
    gstart = jnp.arange(NT, dtype=jnp.int32) * TILE_G
    tile_expert = jnp.minimum(
        jnp.sum((gstart[:, None] >= ends[None, :]).astype(jnp.int32), axis=1),
        E - 1).astype(jnp.int32)
    n_active = (jnp.sum(padded) // TILE_G).astype(jnp.int32).reshape(1)

    x_sorted = _sc_scatter_x(x, posk)

    eo_sorted = pl.pallas_call(
        _ffn_body,
        grid_spec=pltpu.PrefetchScalarGridSpec(
            num_scalar_prefetch=2,
            grid=(NT,),
            in_specs=[
                pl.BlockSpec((TILE_G, D), lambda g, te, na: (g, 0)),
                pl.BlockSpec((1, D, H), lambda g, te, na: (te[g], 0, 0)),
                pl.BlockSpec((1, 1, H), lambda g, te, na: (te[g], 0, 0)),
                pl.BlockSpec((1, H, D), lambda g, te, na: (te[g], 0, 0)),
                pl.BlockSpec((1, 1, D), lambda g, te, na: (te[g], 0, 0)),
            ],
            out_specs=pl.BlockSpec((TILE_G, D), lambda g, te, na: (g, 0)),
            scratch_shapes=[
                pltpu.VMEM((D, H), jnp.bfloat16),
                pltpu.VMEM((H, D), jnp.bfloat16),
            ],
        ),
        out_shape=jax.ShapeDtypeStruct((NP, D), jnp.float32),
    )(tile_expert, n_active, x_sorted, W1, b1.reshape(E, 1, H),
      W2, b2.reshape(E, 1, D))

    g_rows = _sc_gather_back(eo_sorted, posk)

    out = pl.pallas_call(
        _combine_body,
        grid=(T // TILE_T,),
        in_specs=[
            pl.BlockSpec((TILE_T, D), lambda t: (t, 0)),
            pl.BlockSpec((TILE_T, D), lambda t: (t, 0)),
            pl.BlockSpec((TILE_T, D), lambda t: (t + T // TILE_T, 0)),
            pl.BlockSpec((TILE_T, K), lambda t: (t, 0)),
            pl.BlockSpec((1, D), lambda t: (0, 0)),
            pl.BlockSpec((1, D), lambda t: (0, 0)),
            pl.BlockSpec(memory_space=pltpu.SMEM),
        ],
        out_specs=pl.BlockSpec((TILE_T, D), lambda t: (t, 0)),
        out_shape=jax.ShapeDtypeStruct((T, D), jnp.float32),
    )(x, g_rows, g_rows, w_top, gamma.reshape(1, D), beta.reshape(1, D),
      gate_scale.reshape(1))
    return out


# speedup vs baseline: 1.0846x; 1.0846x over previous
import functools

import jax
import jax.numpy as jnp
from jax import lax
from jax.experimental import pallas as pl
from jax.experimental.pallas import tpu as pltpu
from jax.experimental.pallas import tpu_sc as plsc

T = 2048
D = 768
H = 4 * D
E = 8
K = 2
A = T * K

TILE_G = 640
NT = -(-A // TILE_G) + E
NP = NT * TILE_G
TILE_T = 256

NC, NS = 2, 16
NW = NC * NS
CH = T // NW

_sc_mesh = plsc.VectorSubcoreMesh(core_axis_name="c", subcore_axis_name="s")


def _router_body(x_ref, wr_ref, w_ref, idx_ref):
    logits = jnp.dot(x_ref[...], wr_ref[...], preferred_element_type=jnp.float32)
    m = jnp.max(logits, axis=-1, keepdims=True)
    el = jnp.exp(logits - m)
    probs = el / jnp.sum(el, axis=-1, keepdims=True)
    lane = jax.lax.broadcasted_iota(jnp.int32, probs.shape, 1)
    p1 = jnp.max(probs, axis=-1, keepdims=True)
    i1 = jnp.min(jnp.where(probs == p1, lane, E), axis=-1, keepdims=True)
    pm = jnp.where(lane == i1, -1.0, probs)
    p2 = jnp.max(pm, axis=-1, keepdims=True)
    i2 = jnp.min(jnp.where(pm == p2, lane, E), axis=-1, keepdims=True)
    w_ref[...] = jnp.concatenate([p1, p2], axis=-1)
    idx_ref[...] = jnp.concatenate([i1, i2], axis=-1)


@functools.partial(
    pl.kernel, mesh=_sc_mesh,
    out_type=jax.ShapeDtypeStruct((NP, D), jnp.float32),
    scratch_types=[
        pltpu.VMEM((CH,), jnp.int32),
        pltpu.VMEM((CH,), jnp.int32),
        pltpu.VMEM((CH, D), jnp.float32),
        pltpu.SemaphoreType.DMA,
    ],
)
def _sc_scatter_x(x_hbm, posk_hbm, xs_hbm, idx0_v, idx1_v, rows_v, sem):
    wid = lax.axis_index("s") * NC + lax.axis_index("c")
    base = wid * CH
    pltpu.sync_copy(posk_hbm.at[0, pl.ds(base, CH)], idx0_v)
    pltpu.sync_copy(posk_hbm.at[1, pl.ds(base, CH)], idx1_v)
    pltpu.sync_copy(x_hbm.at[pl.ds(base, CH)], rows_v)
    pltpu.async_copy(rows_v, xs_hbm.at[idx0_v], sem).wait()
    pltpu.async_copy(rows_v, xs_hbm.at[idx1_v], sem).wait()


@functools.partial(
    pl.kernel, mesh=_sc_mesh,
    out_type=jax.ShapeDtypeStruct((2 * T, D), jnp.float32),
    scratch_types=[
        pltpu.VMEM((CH,), jnp.int32),
        pltpu.VMEM((CH, D), jnp.float32),
        pltpu.SemaphoreType.DMA,
    ],
)
def _sc_gather_back(eo_hbm, posk_hbm, g_hbm, idx_v, rows_v, sem):
    wid = lax.axis_index("s") * NC + lax.axis_index("c")
    base = wid * CH
    pltpu.sync_copy(posk_hbm.at[0, pl.ds(base, CH)], idx_v)
    pltpu.async_copy(eo_hbm.at[idx_v], rows_v, sem).wait()
    pltpu.sync_copy(rows_v, g_hbm.at[pl.ds(base, CH)])
    pltpu.sync_copy(posk_hbm.at[1, pl.ds(base, CH)], idx_v)
    pltpu.async_copy(eo_hbm.at[idx_v], rows_v, sem).wait()
    pltpu.sync_copy(rows_v, g_hbm.at[pl.ds(T + base, CH)])


def _ffn_body(te_ref, nact_ref, x_ref, w1_ref, b1_ref, w2_ref, b2_ref, out_ref,
              w1b_ref, w2b_ref):
    g = pl.program_id(0)
    prev_e = jnp.where(g == 0, -1, te_ref[jnp.maximum(g - 1, 0)])

    @pl.when((g < nact_ref[0]) & (te_ref[g] != prev_e))
    def _cast_weights():
        w1b_ref[...] = w1_ref[0].astype(jnp.bfloat16)
        w2b_ref[...] = w2_ref[0].astype(jnp.bfloat16)

    @pl.when(g < nact_ref[0])
    def _compute():
        xb = x_ref[...].astype(jnp.bfloat16)
        h = jnp.dot(xb, w1b_ref[...], preferred_element_type=jnp.float32)
        h = h + b1_ref[0]
        h = h * (1.0 / (1.0 + jnp.exp(-h)))
        out_ref[...] = jnp.dot(h.astype(jnp.bfloat16), w2b_ref[...],
                               preferred_element_type=jnp.float32) + b2_ref[0]


def _combine_body(x_ref, g1_ref, g2_ref, w_ref, gamma_ref, beta_ref, gs_ref,
                  out_ref):
    w = w_ref[...]
    out = w[:, 0:1] * g1_ref[...] + w[:, 1:2] * g2_ref[...]
    y = x_ref[...] + out * gs_ref[0]
    mu = jnp.mean(y, axis=-1, keepdims=True)
    yc = y - mu
    var = jnp.mean(yc * yc, axis=-1, keepdims=True)
    out_ref[...] = yc * jax.lax.rsqrt(var + 1e-5) * gamma_ref[...] + beta_ref[...]


@jax.jit
def kernel(x, W_r, W1, b1, W2, b2, gamma, beta, gate_scale):
    w_top, idx_top = pl.pallas_call(
        _router_body,
        grid=(T // TILE_T,),
        in_specs=[
            pl.BlockSpec((TILE_T, D), lambda t: (t, 0)),
            pl.BlockSpec((D, E), lambda t: (0, 0)),
        ],
        out_specs=[
            pl.BlockSpec((TILE_T, K), lambda t: (t, 0)),
            pl.BlockSpec((TILE_T, K), lambda t: (t, 0)),
        ],
        out_shape=[
            jax.ShapeDtypeStruct((T, K), jnp.float32),
            jax.ShapeDtypeStruct((T, K), jnp.int32),
        ],
    )(x, W_r)

    ef = idx_top.reshape(-1)
    oh = (ef[:, None] == jnp.arange(E, dtype=jnp.int32)[None, :]).astype(jnp.int32)
    csum = jnp.cumsum(oh, axis=0)
    counts = csum[-1]
    rank = jnp.sum(csum * oh, axis=1) - 1
    padded = ((counts + TILE_G - 1) // TILE_G) * TILE_G
    base = jnp.concatenate([jnp.zeros((1,), jnp.int32),
                            jnp.cumsum(padded)[:-1].astype(jnp.int32)])
    pos = base[ef] + rank
    posk = pos.reshape(T, K).T
    ends = base + padded
    gstart = jnp.arange(NT, dtype=jnp.int32) * TILE_G
    tile_expert = jnp.minimum(
        jnp.sum((gstart[:, None] >= ends[None, :]).astype(jnp.int32), axis=1),
        E - 1).astype(jnp.int32)
    n_active = (jnp.sum(padded) // TILE_G).astype(jnp.int32).reshape(1)

    x_sorted = _sc_scatter_x(x, posk)

    eo_sorted = pl.pallas_call(
        _ffn_body,
        grid_spec=pltpu.PrefetchScalarGridSpec(
            num_scalar_prefetch=2,
            grid=(NT,),
            in_specs=[
                pl.BlockSpec((TILE_G, D), lambda g, te, na: (g, 0)),
                pl.BlockSpec((1, D, H), lambda g, te, na: (te[g], 0, 0)),
                pl.BlockSpec((1, 1, H), lambda g, te, na: (te[g], 0, 0)),
                pl.BlockSpec((1, H, D), lambda g, te, na: (te[g], 0, 0)),
                pl.BlockSpec((1, 1, D), lambda g, te, na: (te[g], 0, 0)),
            ],
            out_specs=pl.BlockSpec((TILE_G, D), lambda g, te, na: (g, 0)),
            scratch_shapes=[
                pltpu.VMEM((D, H), jnp.bfloat16),
                pltpu.VMEM((H, D), jnp.bfloat16),
            ],
        ),
        out_shape=jax.ShapeDtypeStruct((NP, D), jnp.float32),
    )(tile_expert, n_active, x_sorted, W1, b1.reshape(E, 1, H),
      W2, b2.reshape(E, 1, D))

    g_rows = _sc_gather_back(eo_sorted, posk)

    out = pl.pallas_call(
        _combine_body,
        grid=(T // TILE_T,),
        in_specs=[
            pl.BlockSpec((TILE_T, D), lambda t: (t, 0)),
            pl.BlockSpec((TILE_T, D), lambda t: (t, 0)),
            pl.BlockSpec((TILE_T, D), lambda t: (t + T // TILE_T, 0)),
            pl.BlockSpec((TILE_T, K), lambda t: (t, 0)),
            pl.BlockSpec((1, D), lambda t: (0, 0)),
            pl.BlockSpec((1, D), lambda t: (0, 0)),
            pl.BlockSpec(memory_space=pltpu.SMEM),
        ],
        out_specs=pl.BlockSpec((TILE_T, D), lambda t: (t, 0)),
        out_shape=jax.ShapeDtypeStruct((T, D), jnp.float32),
    )(x, g_rows, g_rows, w_top, gamma.reshape(1, D), beta.reshape(1, D),
      gate_scale.reshape(1))
    return out


# speedup vs baseline: 1.1111x; 1.0244x over previous
import functools

import jax
import jax.numpy as jnp
from jax import lax
from jax.experimental import pallas as pl
from jax.experimental.pallas import tpu as pltpu
from jax.experimental.pallas import tpu_sc as plsc

T = 2048
D = 768
H = 4 * D
E = 8
K = 2
A = T * K

TILE_G = 640
NT = -(-A // TILE_G) + E
NP = NT * TILE_G
TILE_T = 256

NC, NS = 2, 16
NW = NC * NS
CH = T // NW

_sc_mesh = plsc.VectorSubcoreMesh(core_axis_name="c", subcore_axis_name="s")


def _router_body(x_ref, wr_ref, w_ref, idx_ref):
    logits = jnp.dot(x_ref[...], wr_ref[...], preferred_element_type=jnp.float32)
    m = jnp.max(logits, axis=-1, keepdims=True)
    el = jnp.exp(logits - m)
    probs = el / jnp.sum(el, axis=-1, keepdims=True)
    lane = jax.lax.broadcasted_iota(jnp.int32, probs.shape, 1)
    p1 = jnp.max(probs, axis=-1, keepdims=True)
    i1 = jnp.min(jnp.where(probs == p1, lane, E), axis=-1, keepdims=True)
    pm = jnp.where(lane == i1, -1.0, probs)
    p2 = jnp.max(pm, axis=-1, keepdims=True)
    i2 = jnp.min(jnp.where(pm == p2, lane, E), axis=-1, keepdims=True)
    w_ref[...] = jnp.concatenate([p1, p2], axis=-1)
    idx_ref[...] = jnp.concatenate([i1, i2], axis=-1)


@functools.partial(
    pl.kernel, mesh=_sc_mesh,
    out_type=jax.ShapeDtypeStruct((NP, D), jnp.float32),
    scratch_types=[
        pltpu.VMEM((CH,), jnp.int32),
        pltpu.VMEM((CH,), jnp.int32),
        pltpu.VMEM((CH, D), jnp.float32),
        pltpu.SemaphoreType.DMA,
    ],
)
def _sc_scatter_x(x_hbm, posk_hbm, xs_hbm, idx0_v, idx1_v, rows_v, sem):
    wid = lax.axis_index("s") * NC + lax.axis_index("c")
    base = wid * CH
    pltpu.sync_copy(posk_hbm.at[0, pl.ds(base, CH)], idx0_v)
    pltpu.sync_copy(posk_hbm.at[1, pl.ds(base, CH)], idx1_v)
    pltpu.sync_copy(x_hbm.at[pl.ds(base, CH)], rows_v)
    pltpu.async_copy(rows_v, xs_hbm.at[idx0_v], sem).wait()
    pltpu.async_copy(rows_v, xs_hbm.at[idx1_v], sem).wait()


@functools.partial(
    pl.kernel, mesh=_sc_mesh,
    out_type=jax.ShapeDtypeStruct((2 * T, D), jnp.float32),
    scratch_types=[
        pltpu.VMEM((CH,), jnp.int32),
        pltpu.VMEM((CH, D), jnp.float32),
        pltpu.SemaphoreType.DMA,
    ],
)
def _sc_gather_back(eo_hbm, posk_hbm, g_hbm, idx_v, rows_v, sem):
    wid = lax.axis_index("s") * NC + lax.axis_index("c")
    base = wid * CH
    pltpu.sync_copy(posk_hbm.at[0, pl.ds(base, CH)], idx_v)
    pltpu.async_copy(eo_hbm.at[idx_v], rows_v, sem).wait()
    pltpu.sync_copy(rows_v, g_hbm.at[pl.ds(base, CH)])
    pltpu.sync_copy(posk_hbm.at[1, pl.ds(base, CH)], idx_v)
    pltpu.async_copy(eo_hbm.at[idx_v], rows_v, sem).wait()
    pltpu.sync_copy(rows_v, g_hbm.at[pl.ds(T + base, CH)])


def _ffn_body(te_ref, nact_ref, x_ref, w1_ref, b1_ref, w2_ref, b2_ref, out_ref):
    g = pl.program_id(0)

    @pl.when(g < nact_ref[0])
    def _compute():
        x = x_ref[...]
        h = jax.lax.dot_general(x, w1_ref[0], (((1,), (0,)), ((), ())),
                                precision=jax.lax.Precision.DEFAULT,
                                preferred_element_type=jnp.float32)
        h = h + b1_ref[0]
        h = h * (1.0 / (1.0 + jnp.exp(-h)))
        eo = jax.lax.dot_general(h, w2_ref[0], (((1,), (0,)), ((), ())),
                                 precision=jax.lax.Precision.DEFAULT,
                                 preferred_element_type=jnp.float32)
        out_ref[...] = eo + b2_ref[0]


def _combine_body(x_ref, g1_ref, g2_ref, w_ref, gamma_ref, beta_ref, gs_ref,
                  out_ref):
    w = w_ref[...]
    out = w[:, 0:1] * g1_ref[...] + w[:, 1:2] * g2_ref[...]
    y = x_ref[...] + out * gs_ref[0]
    mu = jnp.mean(y, axis=-1, keepdims=True)
    yc = y - mu
    var = jnp.mean(yc * yc, axis=-1, keepdims=True)
    out_ref[...] = yc * jax.lax.rsqrt(var + 1e-5) * gamma_ref[...] + beta_ref[...]


@jax.jit
def kernel(x, W_r, W1, b1, W2, b2, gamma, beta, gate_scale):
    w_top, idx_top = pl.pallas_call(
        _router_body,
        grid=(T // TILE_T,),
        in_specs=[
            pl.BlockSpec((TILE_T, D), lambda t: (t, 0)),
            pl.BlockSpec((D, E), lambda t: (0, 0)),
        ],
        out_specs=[
            pl.BlockSpec((TILE_T, K), lambda t: (t, 0)),
            pl.BlockSpec((TILE_T, K), lambda t: (t, 0)),
        ],
        out_shape=[
            jax.ShapeDtypeStruct((T, K), jnp.float32),
            jax.ShapeDtypeStruct((T, K), jnp.int32),
        ],
    )(x, W_r)

    ef = idx_top.reshape(-1)
    oh = (ef[:, None] == jnp.arange(E, dtype=jnp.int32)[None, :]).astype(jnp.int32)
    csum = jnp.cumsum(oh, axis=0)
    counts = csum[-1]
    rank = jnp.sum(csum * oh, axis=1) - 1
    padded = ((counts + TILE_G - 1) // TILE_G) * TILE_G
    base = jnp.concatenate([jnp.zeros((1,), jnp.int32),
                            jnp.cumsum(padded)[:-1].astype(jnp.int32)])
    pos = base[ef] + rank
    posk = pos.reshape(T, K).T
    ends = base + padded
    gstart = jnp.arange(NT, dtype=jnp.int32) * TILE_G
    tile_expert = jnp.minimum(
        jnp.sum((gstart[:, None] >= ends[None, :]).astype(jnp.int32), axis=1),
        E - 1).astype(jnp.int32)
    n_active = (jnp.sum(padded) // TILE_G).astype(jnp.int32).reshape(1)

    x_sorted = _sc_scatter_x(x, posk)

    eo_sorted = pl.pallas_call(
        _ffn_body,
        grid_spec=pltpu.PrefetchScalarGridSpec(
            num_scalar_prefetch=2,
            grid=(NT,),
            in_specs=[
                pl.BlockSpec((TILE_G, D), lambda g, te, na: (g, 0)),
                pl.BlockSpec((1, D, H), lambda g, te, na: (te[g], 0, 0)),
                pl.BlockSpec((1, 1, H), lambda g, te, na: (te[g], 0, 0)),
                pl.BlockSpec((1, H, D), lambda g, te, na: (te[g], 0, 0)),
                pl.BlockSpec((1, 1, D), lambda g, te, na: (te[g], 0, 0)),
            ],
            out_specs=pl.BlockSpec((TILE_G, D), lambda g, te, na: (g, 0)),
        ),
        out_shape=jax.ShapeDtypeStruct((NP, D), jnp.float32),
    )(tile_expert, n_active, x_sorted, W1, b1.reshape(E, 1, H),
      W2, b2.reshape(E, 1, D))

    g_rows = _sc_gather_back(eo_sorted, posk)

    out = pl.pallas_call(
        _combine_body,
        grid=(T // TILE_T,),
        in_specs=[
            pl.BlockSpec((TILE_T, D), lambda t: (t, 0)),
            pl.BlockSpec((TILE_T, D), lambda t: (t, 0)),
            pl.BlockSpec((TILE_T, D), lambda t: (t + T // TILE_T, 0)),
            pl.BlockSpec((TILE_T, K), lambda t: (t, 0)),
            pl.BlockSpec((1, D), lambda t: (0, 0)),
            pl.BlockSpec((1, D), lambda t: (0, 0)),
            pl.BlockSpec(memory_space=pltpu.SMEM),
        ],
        out_specs=pl.BlockSpec((TILE_T, D), lambda t: (t, 0)),
        out_shape=jax.ShapeDtypeStruct((T, D), jnp.float32),
    )(x, g_rows, g_rows, w_top, gamma.reshape(1, D), beta.reshape(1, D),
      gate_scale.reshape(1))
    return out


# speedup vs baseline: 1.1301x; 1.0171x over previous
import functools

import jax
import jax.numpy as jnp
from jax import lax
from jax.experimental import pallas as pl
from jax.experimental.pallas import tpu as pltpu
from jax.experimental.pallas import tpu_sc as plsc

T = 2048
D = 768
H = 4 * D
E = 8
K = 2
A = T * K

TILE_G = 576
NT = -(-A // TILE_G) + E
NP = NT * TILE_G
TILE_T = 256

NC, NS = 2, 16
NW = NC * NS
CH = T // NW

_sc_mesh = plsc.VectorSubcoreMesh(core_axis_name="c", subcore_axis_name="s")


def _router_body(x_ref, wr_ref, w_ref, idx_ref):
    logits = jnp.dot(x_ref[...], wr_ref[...], preferred_element_type=jnp.float32)
    m = jnp.max(logits, axis=-1, keepdims=True)
    el = jnp.exp(logits - m)
    probs = el / jnp.sum(el, axis=-1, keepdims=True)
    lane = jax.lax.broadcasted_iota(jnp.int32, probs.shape, 1)
    p1 = jnp.max(probs, axis=-1, keepdims=True)
    i1 = jnp.min(jnp.where(probs == p1, lane, E), axis=-1, keepdims=True)
    pm = jnp.where(lane == i1, -1.0, probs)
    p2 = jnp.max(pm, axis=-1, keepdims=True)
    i2 = jnp.min(jnp.where(pm == p2, lane, E), axis=-1, keepdims=True)
    w_ref[...] = jnp.concatenate([p1, p2], axis=-1)
    idx_ref[...] = jnp.concatenate([i1, i2], axis=-1)


@functools.partial(
    pl.kernel, mesh=_sc_mesh,
    out_type=jax.ShapeDtypeStruct((NP, D), jnp.float32),
    scratch_types=[
        pltpu.VMEM((CH,), jnp.int32),
        pltpu.VMEM((CH,), jnp.int32),
        pltpu.VMEM((CH, D), jnp.float32),
        pltpu.SemaphoreType.DMA,
    ],
)
def _sc_scatter_x(x_hbm, posk_hbm, xs_hbm, idx0_v, idx1_v, rows_v, sem):
    wid = lax.axis_index("s") * NC + lax.axis_index("c")
    base = wid * CH
    pltpu.sync_copy(posk_hbm.at[0, pl.ds(base, CH)], idx0_v)
    pltpu.sync_copy(posk_hbm.at[1, pl.ds(base, CH)], idx1_v)
    pltpu.sync_copy(x_hbm.at[pl.ds(base, CH)], rows_v)
    pltpu.async_copy(rows_v, xs_hbm.at[idx0_v], sem).wait()
    pltpu.async_copy(rows_v, xs_hbm.at[idx1_v], sem).wait()


@functools.partial(
    pl.kernel, mesh=_sc_mesh,
    out_type=jax.ShapeDtypeStruct((2 * T, D), jnp.float32),
    scratch_types=[
        pltpu.VMEM((CH,), jnp.int32),
        pltpu.VMEM((CH, D), jnp.float32),
        pltpu.SemaphoreType.DMA,
    ],
)
def _sc_gather_back(eo_hbm, posk_hbm, g_hbm, idx_v, rows_v, sem):
    wid = lax.axis_index("s") * NC + lax.axis_index("c")
    base = wid * CH
    pltpu.sync_copy(posk_hbm.at[0, pl.ds(base, CH)], idx_v)
    pltpu.async_copy(eo_hbm.at[idx_v], rows_v, sem).wait()
    pltpu.sync_copy(rows_v, g_hbm.at[pl.ds(base, CH)])
    pltpu.sync_copy(posk_hbm.at[1, pl.ds(base, CH)], idx_v)
    pltpu.async_copy(eo_hbm.at[idx_v], rows_v, sem).wait()
    pltpu.sync_copy(rows_v, g_hbm.at[pl.ds(T + base, CH)])


def _ffn_body(te_ref, nact_ref, x_ref, w1_ref, b1_ref, w2_ref, b2_ref, out_ref):
    g = pl.program_id(0)

    @pl.when(g < nact_ref[0])
    def _compute():
        x = x_ref[...]
        h = jax.lax.dot_general(x, w1_ref[0], (((1,), (0,)), ((), ())),
                                precision=jax.lax.Precision.DEFAULT,
                                preferred_element_type=jnp.float32)
        h = h + b1_ref[0]
        h = h * (1.0 / (1.0 + jnp.exp(-h)))
        eo = jax.lax.dot_general(h, w2_ref[0], (((1,), (0,)), ((), ())),
                                 precision=jax.lax.Precision.DEFAULT,
                                 preferred_element_type=jnp.float32)
        out_ref[...] = eo + b2_ref[0]


def _combine_body(x_ref, g1_ref, g2_ref, w_ref, gamma_ref, beta_ref, gs_ref,
                  out_ref):
    w = w_ref[...]
    out = w[:, 0:1] * g1_ref[...] + w[:, 1:2] * g2_ref[...]
    y = x_ref[...] + out * gs_ref[0]
    mu = jnp.mean(y, axis=-1, keepdims=True)
    yc = y - mu
    var = jnp.mean(yc * yc, axis=-1, keepdims=True)
    out_ref[...] = yc * jax.lax.rsqrt(var + 1e-5) * gamma_ref[...] + beta_ref[...]


@jax.jit
def kernel(x, W_r, W1, b1, W2, b2, gamma, beta, gate_scale):
    w_top, idx_top = pl.pallas_call(
        _router_body,
        grid=(T // TILE_T,),
        in_specs=[
            pl.BlockSpec((TILE_T, D), lambda t: (t, 0)),
            pl.BlockSpec((D, E), lambda t: (0, 0)),
        ],
        out_specs=[
            pl.BlockSpec((TILE_T, K), lambda t: (t, 0)),
            pl.BlockSpec((TILE_T, K), lambda t: (t, 0)),
        ],
        out_shape=[
            jax.ShapeDtypeStruct((T, K), jnp.float32),
            jax.ShapeDtypeStruct((T, K), jnp.int32),
        ],
    )(x, W_r)

    ef = idx_top.reshape(-1)
    oh = (ef[:, None] == jnp.arange(E, dtype=jnp.int32)[None, :]).astype(jnp.int32)
    csum = jnp.cumsum(oh, axis=0)
    counts = csum[-1]
    rank = jnp.sum(csum * oh, axis=1) - 1
    padded = ((counts + TILE_G - 1) // TILE_G) * TILE_G
    base = jnp.concatenate([jnp.zeros((1,), jnp.int32),
                            jnp.cumsum(padded)[:-1].astype(jnp.int32)])
    pos = base[ef] + rank
    posk = pos.reshape(T, K).T
    ends = base + padded
    gstart = jnp.arange(NT, dtype=jnp.int32) * TILE_G
    tile_expert = jnp.minimum(
        jnp.sum((gstart[:, None] >= ends[None, :]).astype(jnp.int32), axis=1),
        E - 1).astype(jnp.int32)
    n_active = (jnp.sum(padded) // TILE_G).astype(jnp.int32).reshape(1)

    x_sorted = _sc_scatter_x(x, posk)

    eo_sorted = pl.pallas_call(
        _ffn_body,
        grid_spec=pltpu.PrefetchScalarGridSpec(
            num_scalar_prefetch=2,
            grid=(NT,),
            in_specs=[
                pl.BlockSpec((TILE_G, D), lambda g, te, na: (g, 0)),
                pl.BlockSpec((1, D, H), lambda g, te, na: (te[g], 0, 0)),
                pl.BlockSpec((1, 1, H), lambda g, te, na: (te[g], 0, 0)),
                pl.BlockSpec((1, H, D), lambda g, te, na: (te[g], 0, 0)),
                pl.BlockSpec((1, 1, D), lambda g, te, na: (te[g], 0, 0)),
            ],
            out_specs=pl.BlockSpec((TILE_G, D), lambda g, te, na: (g, 0)),
        ),
        out_shape=jax.ShapeDtypeStruct((NP, D), jnp.float32),
    )(tile_expert, n_active, x_sorted, W1, b1.reshape(E, 1, H),
      W2, b2.reshape(E, 1, D))

    g_rows = _sc_gather_back(eo_sorted, posk)

    out = pl.pallas_call(
        _combine_body,
        grid=(T // TILE_T,),
        in_specs=[
            pl.BlockSpec((TILE_T, D), lambda t: (t, 0)),
            pl.BlockSpec((TILE_T, D), lambda t: (t, 0)),
            pl.BlockSpec((TILE_T, D), lambda t: (t + T // TILE_T, 0)),
            pl.BlockSpec((TILE_T, K), lambda t: (t, 0)),
            pl.BlockSpec((1, D), lambda t: (0, 0)),
            pl.BlockSpec((1, D), lambda t: (0, 0)),
            pl.BlockSpec(memory_space=pltpu.SMEM),
        ],
        out_specs=pl.BlockSpec((TILE_T, D), lambda t: (t, 0)),
        out_shape=jax.ShapeDtypeStruct((T, D), jnp.float32),
    )(x, g_rows, g_rows, w_top, gamma.reshape(1, D), beta.reshape(1, D),
      gate_scale.reshape(1))
    return out


# speedup vs baseline: 1.1658x; 1.0317x over previous
import functools

import jax
import jax.numpy as jnp
from jax import lax
from jax.experimental import pallas as pl
from jax.experimental.pallas import tpu as pltpu
from jax.experimental.pallas import tpu_sc as plsc

T = 2048
D = 768
H = 4 * D
E = 8
K = 2
A = T * K

TILE_G = 576
NT = -(-A // TILE_G) + E
NP = NT * TILE_G
TILE_T = 512

NC, NS = 2, 16
NW = NC * NS
CH = T // NW

_sc_mesh = plsc.VectorSubcoreMesh(core_axis_name="c", subcore_axis_name="s")


def _router_body(x_ref, wr_ref, w_ref, idx_ref):
    logits = jnp.dot(x_ref[...], wr_ref[...], preferred_element_type=jnp.float32)
    m = jnp.max(logits, axis=-1, keepdims=True)
    el = jnp.exp(logits - m)
    probs = el / jnp.sum(el, axis=-1, keepdims=True)
    lane = jax.lax.broadcasted_iota(jnp.int32, probs.shape, 1)
    p1 = jnp.max(probs, axis=-1, keepdims=True)
    i1 = jnp.min(jnp.where(probs == p1, lane, E), axis=-1, keepdims=True)
    pm = jnp.where(lane == i1, -1.0, probs)
    p2 = jnp.max(pm, axis=-1, keepdims=True)
    i2 = jnp.min(jnp.where(pm == p2, lane, E), axis=-1, keepdims=True)
    w_ref[...] = jnp.concatenate([p1, p2], axis=-1)
    idx_ref[...] = jnp.concatenate([i1, i2], axis=-1)


@functools.partial(
    pl.kernel, mesh=_sc_mesh,
    out_type=jax.ShapeDtypeStruct((NP, D), jnp.float32),
    scratch_types=[
        pltpu.VMEM((CH,), jnp.int32),
        pltpu.VMEM((CH,), jnp.int32),
        pltpu.VMEM((CH, D), jnp.float32),
        pltpu.SemaphoreType.DMA,
    ],
)
def _sc_scatter_x(x_hbm, posk_hbm, xs_hbm, idx0_v, idx1_v, rows_v, sem):
    wid = lax.axis_index("s") * NC + lax.axis_index("c")
    base = wid * CH
    pltpu.sync_copy(posk_hbm.at[0, pl.ds(base, CH)], idx0_v)
    pltpu.sync_copy(posk_hbm.at[1, pl.ds(base, CH)], idx1_v)
    pltpu.sync_copy(x_hbm.at[pl.ds(base, CH)], rows_v)
    pltpu.async_copy(rows_v, xs_hbm.at[idx0_v], sem).wait()
    pltpu.async_copy(rows_v, xs_hbm.at[idx1_v], sem).wait()


@functools.partial(
    pl.kernel, mesh=_sc_mesh,
    out_type=jax.ShapeDtypeStruct((2 * T, D), jnp.float32),
    scratch_types=[
        pltpu.VMEM((CH,), jnp.int32),
        pltpu.VMEM((CH, D), jnp.float32),
        pltpu.SemaphoreType.DMA,
    ],
)
def _sc_gather_back(eo_hbm, posk_hbm, g_hbm, idx_v, rows_v, sem):
    wid = lax.axis_index("s") * NC + lax.axis_index("c")
    base = wid * CH
    pltpu.sync_copy(posk_hbm.at[0, pl.ds(base, CH)], idx_v)
    pltpu.async_copy(eo_hbm.at[idx_v], rows_v, sem).wait()
    pltpu.sync_copy(rows_v, g_hbm.at[pl.ds(base, CH)])
    pltpu.sync_copy(posk_hbm.at[1, pl.ds(base, CH)], idx_v)
    pltpu.async_copy(eo_hbm.at[idx_v], rows_v, sem).wait()
    pltpu.sync_copy(rows_v, g_hbm.at[pl.ds(T + base, CH)])


def _ffn_body(te_ref, nact_ref, x_ref, w1_ref, b1_ref, w2_ref, b2_ref, out_ref):
    g = pl.program_id(0)

    @pl.when(g < nact_ref[0])
    def _compute():
        x = x_ref[...]
        h = jax.lax.dot_general(x, w1_ref[0], (((1,), (0,)), ((), ())),
                                precision=jax.lax.Precision.DEFAULT,
                                preferred_element_type=jnp.float32)
        h = h + b1_ref[0]
        h = h * (1.0 / (1.0 + jnp.exp(-h)))
        eo = jax.lax.dot_general(h, w2_ref[0], (((1,), (0,)), ((), ())),
                                 precision=jax.lax.Precision.DEFAULT,
                                 preferred_element_type=jnp.float32)
        out_ref[...] = eo + b2_ref[0]


def _combine_body(x_ref, g1_ref, g2_ref, w_ref, gamma_ref, beta_ref, gs_ref,
                  out_ref):
    w = w_ref[...]
    out = w[:, 0:1] * g1_ref[...] + w[:, 1:2] * g2_ref[...]
    y = x_ref[...] + out * gs_ref[0]
    mu = jnp.mean(y, axis=-1, keepdims=True)
    yc = y - mu
    var = jnp.mean(yc * yc, axis=-1, keepdims=True)
    out_ref[...] = yc * jax.lax.rsqrt(var + 1e-5) * gamma_ref[...] + beta_ref[...]


@jax.jit
def kernel(x, W_r, W1, b1, W2, b2, gamma, beta, gate_scale):
    w_top, idx_top = pl.pallas_call(
        _router_body,
        grid=(T // TILE_T,),
        in_specs=[
            pl.BlockSpec((TILE_T, D), lambda t: (t, 0)),
            pl.BlockSpec((D, E), lambda t: (0, 0)),
        ],
        out_specs=[
            pl.BlockSpec((TILE_T, K), lambda t: (t, 0)),
            pl.BlockSpec((TILE_T, K), lambda t: (t, 0)),
        ],
        out_shape=[
            jax.ShapeDtypeStruct((T, K), jnp.float32),
            jax.ShapeDtypeStruct((T, K), jnp.int32),
        ],
    )(x, W_r)

    ef = idx_top.reshape(-1)
    oh = (ef[:, None] == jnp.arange(E, dtype=jnp.int32)[None, :]).astype(jnp.int32)
    csum = jnp.cumsum(oh, axis=0)
    counts = csum[-1]
    rank = jnp.sum(csum * oh, axis=1) - 1
    padded = ((counts + TILE_G - 1) // TILE_G) * TILE_G
    base = jnp.concatenate([jnp.zeros((1,), jnp.int32),
                            jnp.cumsum(padded)[:-1].astype(jnp.int32)])
    pos = base[ef] + rank
    posk = pos.reshape(T, K).T
    ends = base + padded
    gstart = jnp.arange(NT, dtype=jnp.int32) * TILE_G
    tile_expert = jnp.minimum(
        jnp.sum((gstart[:, None] >= ends[None, :]).astype(jnp.int32), axis=1),
        E - 1).astype(jnp.int32)
    n_active = (jnp.sum(padded) // TILE_G).astype(jnp.int32).reshape(1)

    x_sorted = _sc_scatter_x(x, posk)

    eo_sorted = pl.pallas_call(
        _ffn_body,
        grid_spec=pltpu.PrefetchScalarGridSpec(
            num_scalar_prefetch=2,
            grid=(NT,),
            in_specs=[
                pl.BlockSpec((TILE_G, D), lambda g, te, na: (g, 0)),
                pl.BlockSpec((1, D, H), lambda g, te, na: (te[g], 0, 0)),
                pl.BlockSpec((1, 1, H), lambda g, te, na: (te[g], 0, 0)),
                pl.BlockSpec((1, H, D), lambda g, te, na: (te[g], 0, 0)),
                pl.BlockSpec((1, 1, D), lambda g, te, na: (te[g], 0, 0)),
            ],
            out_specs=pl.BlockSpec((TILE_G, D), lambda g, te, na: (g, 0)),
        ),
        out_shape=jax.ShapeDtypeStruct((NP, D), jnp.float32),
    )(tile_expert, n_active, x_sorted, W1, b1.reshape(E, 1, H),
      W2, b2.reshape(E, 1, D))

    g_rows = _sc_gather_back(eo_sorted, posk)

    out = pl.pallas_call(
        _combine_body,
        grid=(T // TILE_T,),
        in_specs=[
            pl.BlockSpec((TILE_T, D), lambda t: (t, 0)),
            pl.BlockSpec((TILE_T, D), lambda t: (t, 0)),
            pl.BlockSpec((TILE_T, D), lambda t: (t + T // TILE_T, 0)),
            pl.BlockSpec((TILE_T, K), lambda t: (t, 0)),
            pl.BlockSpec((1, D), lambda t: (0, 0)),
            pl.BlockSpec((1, D), lambda t: (0, 0)),
            pl.BlockSpec(memory_space=pltpu.SMEM),
        ],
        out_specs=pl.BlockSpec((TILE_T, D), lambda t: (t, 0)),
        out_shape=jax.ShapeDtypeStruct((T, D), jnp.float32),
    )(x, g_rows, g_rows, w_top, gamma.reshape(1, D), beta.reshape(1, D),
      gate_scale.reshape(1))
    return out
